# Initial kernel scaffold; baseline (speedup 1.0000x reference)
import jax
import jax.numpy as jnp
from jax.experimental import pallas as pl


def kernel(events):
    raise NotImplementedError("write your pallas kernel here")


# speedup vs baseline: 21.7568x; 21.7568x over previous
import functools

import jax
import jax.numpy as jnp
from jax import lax
from jax.experimental import pallas as pl
from jax.experimental.pallas import tpu as pltpu
from jax.experimental.pallas import tpu_sc as plsc

C, H, W = 9, 260, 346
B = 4
N = 4_000_000
CHW = C * H * W
NUM_VOX = B * CHW

NC, NS, L = 2, 16, 16
NB = N // B
EVT_A, EVT_B = 62_504, 62_496
CHUNK = 5_000
NFULL = 12
TAIL_A = EVT_A - NFULL * CHUNK
TAIL_B = EVT_B - NFULL * CHUNK
FULL_G = CHUNK // L
RAG = CHUNK - FULL_G * L
IDX_LEN = (FULL_G + 1) * L
TAIL_G = TAIL_B // L
IDX_TAIL_LEN = (TAIL_G + 1) * L
SEG = 50_608
SEG_LAST = CHW - 15 * SEG
ZB = 8_192
EVW = CHUNK * 5

_mesh = plsc.VectorSubcoreMesh(core_axis_name="c", subcore_axis_name="s")
_params = pltpu.CompilerParams(needs_layout_passes=False)


@functools.partial(
    pl.kernel,
    out_type=jax.ShapeDtypeStruct((NC * NS * L,), jnp.float32),
    mesh=_mesh,
    compiler_params=_params,
    scratch_types=[
        pltpu.VMEM((EVW,), jnp.float32),
        pltpu.VMEM((EVW,), jnp.float32),
        pltpu.VMEM((L,), jnp.float32),
        pltpu.SemaphoreType.DMA,
        pltpu.SemaphoreType.DMA,
    ],
)
def _tmax_kernel(ev_hbm, out_hbm, ev0, ev1, mbuf, sem0, sem1):
    c = lax.axis_index("c")
    s = lax.axis_index("s")
    wid = c * NS + s
    ev_per_tec = N // (NC * NS)
    nchunk = ev_per_tec // CHUNK
    base_w = (c * (N // NC) + s * ev_per_tec) * 5

    evb = [ev0, ev1]
    sems = [sem0, sem1]
    lane = lax.iota(jnp.int32, L)
    tbase = lane * 5 + 2

    descs = [None, None]
    descs[0] = pltpu.async_copy(ev_hbm.at[pl.ds(base_w, EVW)], ev0, sem0)

    acc = jnp.zeros((L,), jnp.float32)
    for k in range(nchunk):
        descs[k % 2].wait()
        if k + 1 < nchunk:
            descs[(k + 1) % 2] = pltpu.async_copy(
                ev_hbm.at[pl.ds(base_w + (k + 1) * EVW, EVW)],
                evb[(k + 1) % 2], sems[(k + 1) % 2])
        buf = evb[k % 2]

        def body(g, carry):
            gidx, a = carry
            t = plsc.load_gather(buf, [gidx])
            return gidx + 5 * L, jnp.maximum(a, t)

        gidx, acc = lax.fori_loop(0, FULL_G, body, (tbase, acc))
        t = plsc.load_gather(buf, [jnp.minimum(gidx, EVW - 1)])
        acc = jnp.maximum(acc, jnp.where(lane < RAG, t, 0.0))

    mbuf[...] = acc
    pltpu.sync_copy(mbuf, out_hbm.at[pl.ds(wid * L, L)])


@functools.partial(
    pl.kernel,
    out_type=jax.ShapeDtypeStruct((NUM_VOX,), jnp.float32),
    mesh=_mesh,
    compiler_params=_params,
    scratch_types=[
        pltpu.VMEM((EVW,), jnp.float32),
        pltpu.VMEM((EVW,), jnp.float32),
        pltpu.VMEM((IDX_LEN,), jnp.int32),
        pltpu.VMEM((IDX_LEN,), jnp.int32),
        pltpu.VMEM((IDX_TAIL_LEN,), jnp.int32),
        pltpu.VMEM((IDX_LEN,), jnp.float32),
        pltpu.VMEM((IDX_TAIL_LEN,), jnp.float32),
        pltpu.VMEM((NC * NS * L,), jnp.float32),
        pltpu.VMEM((ZB,), jnp.float32),
        pltpu.VMEM_SHARED((CHW,), jnp.float32),
        pltpu.SemaphoreType.DMA,
        pltpu.SemaphoreType.DMA,
        pltpu.SemaphoreType.DMA,
        pltpu.SemaphoreType.DMA,
        pltpu.SemaphoreType.DMA,
    ],
)
def _hist_kernel(ev_hbm, part_hbm, out_hbm, ev0, ev1, idx0, idx1, idxt,
                 ones, onest, pmax, zbuf, grid, esem0, esem1, ssem0, ssem1,
                 ssemt):
    c = lax.axis_index("c")
    s = lax.axis_index("s")
    lane = lax.iota(jnp.int32, L)
    evb = [ev0, ev1]
    esems = [esem0, esem1]
    idxb = [idx0, idx1]
    ssems = [ssem0, ssem1]
    xbase = lane * 5

    pltpu.sync_copy(part_hbm, pmax)
    acc = pmax[pl.ds(0, L)]
    for i in range(1, NC * NS):
        acc = jnp.maximum(acc, pmax[pl.ds(i * L, L)])
    tmaxv = jnp.full((L,), jnp.max(acc), jnp.float32)

    def ones_body(g, _):
        ones[pl.ds(g * L, L)] = jnp.ones((L,), jnp.float32)
        return 0

    lax.fori_loop(0, FULL_G, ones_body, 0)
    ones[pl.ds(FULL_G * L, L)] = jnp.where(lane < RAG, 1.0, 0.0)

    def onest_body(g, _):
        onest[pl.ds(g * L, L)] = jnp.ones((L,), jnp.float32)
        return 0

    lax.fori_loop(0, TAIL_G, onest_body, 0)
    onest[pl.ds(TAIL_G * L, L)] = jnp.where(
        (lane < RAG) & (s < 8), 1.0, 0.0)
    idxt[pl.ds(TAIL_G * L, L)] = jnp.zeros((L,), jnp.int32)

    def zero_body(i, _):
        zbuf[pl.ds(i * L, L)] = jnp.zeros((L,), jnp.float32)
        return 0

    lax.fori_loop(0, ZB // L, zero_body, 0)

    tec_start = jnp.where(s < 8, s * EVT_A, 8 * EVT_A + (s - 8) * EVT_B)
    tail_n = jnp.where(s < 8, TAIL_A, TAIL_B)
    seg_start = s * SEG

    def vox_of(buf, gx, clamp):
        gx = jnp.minimum(gx, clamp)
        x = plsc.load_gather(buf, [gx])
        y = plsc.load_gather(buf, [gx + 1])
        t = plsc.load_gather(buf, [gx + 2])
        bi = ((t / tmaxv) * float(C)).astype(jnp.int32)
        bi = jnp.minimum(bi, C - 1)
        xy = (x + y * float(W)).astype(jnp.int32)
        return xy + bi * (H * W)

    def zero_grid():
        for j in range(SEG // ZB):
            pltpu.sync_copy(zbuf, grid.at[pl.ds(seg_start + j * ZB, ZB)])
        ztail, ztail_last = SEG - (SEG // ZB) * ZB, SEG_LAST - (SEG // ZB) * ZB

        @pl.when(s == NS - 1)
        def _():
            pltpu.sync_copy(zbuf.at[pl.ds(0, ztail_last)],
                            grid.at[pl.ds(seg_start + (SEG // ZB) * ZB,
                                          ztail_last)])

        @pl.when(s != NS - 1)
        def _():
            pltpu.sync_copy(zbuf.at[pl.ds(0, ztail)],
                            grid.at[pl.ds(seg_start + (SEG // ZB) * ZB,
                                          ztail)])

    def scatter_pass(p):
        batch = 2 * c + p
        base_w = (batch * NB + tec_start) * 5
        edescs = [None, None]
        sdescs = [None] * NFULL
        edescs[0] = pltpu.async_copy(
            ev_hbm.at[pl.ds(base_w, EVW)], ev0, esem0)

        for k in range(NFULL):
            edescs[k % 2].wait()
            nxt = (k + 1) % 2
            if k + 1 < NFULL:
                edescs[nxt] = pltpu.async_copy(
                    ev_hbm.at[pl.ds(base_w + (k + 1) * EVW, EVW)],
                    evb[nxt], esems[nxt])
            else:
                @pl.when(s < 8)
                def _():
                    pltpu.async_copy(
                        ev_hbm.at[pl.ds(base_w + NFULL * EVW, TAIL_A * 5)],
                        evb[nxt].at[pl.ds(0, TAIL_A * 5)], esems[nxt])

                @pl.when(s >= 8)
                def _():
                    pltpu.async_copy(
                        ev_hbm.at[pl.ds(base_w + NFULL * EVW, TAIL_B * 5)],
                        evb[nxt].at[pl.ds(0, TAIL_B * 5)], esems[nxt])
            if k >= 2:
                sdescs[k - 2].wait()
            buf, idx = evb[k % 2], idxb[k % 2]

            def body(g, gx):
                idx[pl.ds(g * L, L)] = vox_of(buf, gx, EVW - 3)
                return gx + 5 * L

            gx = lax.fori_loop(0, FULL_G, body, xbase)
            vox = vox_of(buf, gx, EVW - 3)
            idx[pl.ds(FULL_G * L, L)] = jnp.where(lane < RAG, vox, 0)
            sdescs[k] = pltpu.async_copy(
                ones, grid.at[idx], ssems[k % 2], add=True)

        @pl.when(s < 8)
        def _():
            pltpu.make_async_copy(
                ev_hbm.at[pl.ds(base_w + NFULL * EVW, TAIL_A * 5)],
                evb[NFULL % 2].at[pl.ds(0, TAIL_A * 5)],
                esems[NFULL % 2]).wait()

        @pl.when(s >= 8)
        def _():
            pltpu.make_async_copy(
                ev_hbm.at[pl.ds(base_w + NFULL * EVW, TAIL_B * 5)],
                evb[NFULL % 2].at[pl.ds(0, TAIL_B * 5)],
                esems[NFULL % 2]).wait()
        buf = evb[NFULL % 2]

        def tbody(g, gx):
            idxt[pl.ds(g * L, L)] = vox_of(buf, gx, TAIL_B * 5 - 3)
            return gx + 5 * L

        gx = lax.fori_loop(0, TAIL_G, tbody, xbase)

        @pl.when(s < 8)
        def _():
            vox = vox_of(buf, gx, TAIL_A * 5 - 3)
            idxt[pl.ds(TAIL_G * L, L)] = jnp.where(lane < RAG, vox, 0)

        tdesc = pltpu.async_copy(onest, grid.at[idxt], ssemt, add=True)
        sdescs[NFULL - 2].wait()
        sdescs[NFULL - 1].wait()
        tdesc.wait()

    def writeout(p):
        batch = 2 * c + p
        out_base = batch * CHW + seg_start
        nfull = SEG // EVW
        wtail = SEG - nfull * EVW
        wtail_last = SEG_LAST - nfull * EVW
        wdescs = [None, None]
        for j in range(nfull):
            bb = evb[j % 2]
            pltpu.sync_copy(grid.at[pl.ds(seg_start + j * EVW, EVW)], bb)
            wdescs[j % 2] = pltpu.async_copy(
                bb, out_hbm.at[pl.ds(out_base + j * EVW, EVW)],
                esems[j % 2])
        bb = evb[nfull % 2]
        wdescs[nfull % 2].wait()

        @pl.when(s == NS - 1)
        def _():
            pltpu.sync_copy(grid.at[pl.ds(seg_start + nfull * EVW,
                                          wtail_last)],
                            bb.at[pl.ds(0, wtail_last)])
            pltpu.sync_copy(bb.at[pl.ds(0, wtail_last)],
                            out_hbm.at[pl.ds(out_base + nfull * EVW,
                                             wtail_last)])

        @pl.when(s != NS - 1)
        def _():
            pltpu.sync_copy(grid.at[pl.ds(seg_start + nfull * EVW, wtail)],
                            bb.at[pl.ds(0, wtail)])
            pltpu.sync_copy(bb.at[pl.ds(0, wtail)],
                            out_hbm.at[pl.ds(out_base + nfull * EVW,
                                             wtail)])
        wdescs[(nfull + 1) % 2].wait()

    for p in range(2):
        zero_grid()
        plsc.subcore_barrier()
        scatter_pass(p)
        plsc.subcore_barrier()
        writeout(p)


@jax.jit
def kernel(events):
    ev_flat = events.reshape(-1)
    partials = _tmax_kernel(ev_flat)
    grid = _hist_kernel(ev_flat, partials)
    return grid.reshape(-1, C, H, W)
